# baseline (speedup 1.0000x reference)
import functools

import numpy as np
import jax
import jax.numpy as jnp
from jax import lax
from jax.experimental import pallas as pl
from jax.experimental.pallas import tpu as pltpu
from jax.experimental.pallas import tpu_sc as plsc

D_MODEL = 768
MAX_LEN = 5000

NUM_CORES = 2
NUM_SUBCORES = 16
NUM_WORKERS = NUM_CORES * NUM_SUBCORES
LANES = 16
STEP_POS = 16


def _pe_table(time_steps: int) -> np.ndarray:
    half_dim = D_MODEL // 2
    pe = np.zeros((D_MODEL, MAX_LEN), dtype=np.float64)
    pos = np.arange(MAX_LEN)
    freq = 10000 ** (2 * np.arange(half_dim) / D_MODEL)
    pos_freq = pos.reshape((1, -1)) / freq.reshape((-1, 1))
    pe[:half_dim, :] = np.sin(pos_freq)
    pe[half_dim:, :] = np.cos(pos_freq)
    return pe.T[:time_steps].astype(np.float32)


@functools.partial(jax.jit, static_argnames=("batch", "seq_len"))
def _sc_embed(x_r, pe, table, *, batch, seq_len):
    rows_total = batch * seq_len
    chunk = seq_len // NUM_WORKERS
    steps = chunk // STEP_POS
    rows_per_step = batch * STEP_POS
    scale = float(np.sqrt(np.float32(D_MODEL)))
    vregs_per_row = D_MODEL // LANES

    mesh = plsc.VectorSubcoreMesh(
        core_axis_name="c", subcore_axis_name="s")

    @functools.partial(
        pl.kernel,
        out_type=jax.ShapeDtypeStruct((rows_total, D_MODEL), jnp.float32),
        mesh=mesh,
        scratch_types=[
            pltpu.VMEM((batch * chunk,), jnp.int32),
            pltpu.VMEM((rows_per_step, D_MODEL), jnp.float32),
            pltpu.VMEM((rows_per_step, D_MODEL), jnp.float32),
            pltpu.VMEM((STEP_POS, D_MODEL), jnp.float32),
            pltpu.VMEM((STEP_POS, D_MODEL), jnp.float32),
            pltpu.SemaphoreType.DMA,
            pltpu.SemaphoreType.DMA,
            pltpu.SemaphoreType.DMA,
            pltpu.SemaphoreType.DMA,
            pltpu.SemaphoreType.DMA,
            pltpu.SemaphoreType.DMA,
        ],
    )
    def k(xr_hbm, pe_hbm, table_hbm, out_hbm,
          idx_all, rows0, rows1, pe0, pe1, gg0, gg1, gp0, gp1, gs0, gs1):
        rows_v = (rows0, rows1)
        pe_v = (pe0, pe1)
        sem_g = (gg0, gg1)
        sem_p = (gp0, gp1)
        sem_s = (gs0, gs1)

        wid = lax.axis_index("s") * NUM_CORES + lax.axis_index("c")
        t0 = wid * chunk

        pltpu.sync_copy(xr_hbm.at[pl.ds(wid * batch * chunk, batch * chunk)],
                        idx_all)

        def start_step(s, buf):
            g = pltpu.async_copy(
                table_hbm.at[idx_all.at[pl.ds(s * rows_per_step,
                                              rows_per_step)]],
                rows_v[buf], sem_g[buf])
            p = pltpu.async_copy(
                pe_hbm.at[pl.ds(t0 + s * STEP_POS, STEP_POS)],
                pe_v[buf], sem_p[buf])
            return g, p

        def compute(buf):
            def row_body(r, _):
                for c in range(vregs_per_row):
                    sl = pl.ds(c * LANES, LANES)
                    p = pe_v[buf][r, sl]
                    for b in range(batch):
                        rv = rows_v[buf].at[b * STEP_POS + r]
                        rv[sl] = rv[sl] * scale + p
                return 0
            lax.fori_loop(0, STEP_POS, row_body, 0)

        def start_stores(s, buf):
            st = []
            for b in range(batch):
                st.append(pltpu.async_copy(
                    rows_v[buf].at[pl.ds(b * STEP_POS, STEP_POS)],
                    out_hbm.at[pl.ds(b * seq_len + t0 + s * STEP_POS,
                                     STEP_POS)],
                    sem_s[buf]))
            return st

        inflight = [None, None]
        stores = [None, None]
        inflight[0] = start_step(0, 0)
        for s in range(steps):
            buf = s % 2
            if s + 1 < steps:
                nbuf = (s + 1) % 2
                if stores[nbuf] is not None:
                    for st in stores[nbuf]:
                        st.wait()
                    stores[nbuf] = None
                inflight[nbuf] = start_step(s + 1, nbuf)
            g, p = inflight[buf]
            g.wait()
            p.wait()
            compute(buf)
            stores[buf] = start_stores(s, buf)
        for side in stores:
            if side is not None:
                for st in side:
                    st.wait()

    return k(x_r, pe, table)


def kernel(x, table):
    batch, seq_len = x.shape
    chunk = seq_len // NUM_WORKERS
    steps = chunk // STEP_POS
    x_r = jnp.transpose(
        x.reshape(batch, NUM_WORKERS, steps, STEP_POS),
        (1, 2, 0, 3)).reshape(-1)
    pe = jnp.asarray(_pe_table(seq_len))
    out = _sc_embed(x_r, pe, table, batch=batch, seq_len=seq_len)
    return out.reshape(batch, seq_len, D_MODEL)


# speedup vs baseline: 1.2744x; 1.2744x over previous
import functools

import numpy as np
import jax
import jax.numpy as jnp
from jax import lax
from jax.experimental import pallas as pl
from jax.experimental.pallas import tpu as pltpu
from jax.experimental.pallas import tpu_sc as plsc

D_MODEL = 768
MAX_LEN = 5000

NUM_CORES = 2
NUM_SUBCORES = 16
NUM_WORKERS = NUM_CORES * NUM_SUBCORES
LANES = 16
STEP_POS = 16


def _pe_table(time_steps: int) -> np.ndarray:
    half_dim = D_MODEL // 2
    pe = np.zeros((D_MODEL, MAX_LEN), dtype=np.float64)
    pos = np.arange(MAX_LEN)
    freq = 10000 ** (2 * np.arange(half_dim) / D_MODEL)
    pos_freq = pos.reshape((1, -1)) / freq.reshape((-1, 1))
    pe[:half_dim, :] = np.sin(pos_freq)
    pe[half_dim:, :] = np.cos(pos_freq)
    return pe.T[:time_steps].astype(np.float32)


@functools.partial(jax.jit, static_argnames=("batch", "seq_len"))
def _sc_embed(x_r, pe, table, *, batch, seq_len):
    rows_total = batch * seq_len
    chunk = seq_len // NUM_WORKERS
    steps = chunk // STEP_POS
    rows_per_step = batch * STEP_POS
    scale = float(np.sqrt(np.float32(D_MODEL)))
    vregs_per_row = D_MODEL // LANES

    mesh = plsc.VectorSubcoreMesh(
        core_axis_name="c", subcore_axis_name="s")

    @functools.partial(
        pl.kernel,
        out_type=jax.ShapeDtypeStruct((rows_total, D_MODEL), jnp.float32),
        mesh=mesh,
        scratch_types=[
            pltpu.VMEM((chunk // STEP_POS, batch * STEP_POS), jnp.int32),
            pltpu.VMEM((rows_per_step, D_MODEL), jnp.float32),
            pltpu.VMEM((rows_per_step, D_MODEL), jnp.float32),
            pltpu.VMEM((STEP_POS, D_MODEL), jnp.float32),
            pltpu.VMEM((STEP_POS, D_MODEL), jnp.float32),
            pltpu.SemaphoreType.DMA,
            pltpu.SemaphoreType.DMA,
            pltpu.SemaphoreType.DMA,
            pltpu.SemaphoreType.DMA,
            pltpu.SemaphoreType.DMA,
            pltpu.SemaphoreType.DMA,
        ],
    )
    def k(xr_hbm, pe_hbm, table_hbm, out_hbm,
          idx_all, rows0, rows1, pe0, pe1, gg0, gg1, gp0, gp1, gs0, gs1):
        rows_v = (rows0, rows1)
        pe_v = (pe0, pe1)
        sem_g = (gg0, gg1)
        sem_p = (gp0, gp1)
        sem_s = (gs0, gs1)

        wid = lax.axis_index("s") * NUM_CORES + lax.axis_index("c")
        t0 = wid * chunk

        pltpu.sync_copy(
            xr_hbm.at[pl.ds(wid * steps, steps)], idx_all)

        def start_step(s, buf):
            g = pltpu.async_copy(
                table_hbm.at[idx_all.at[s]], rows_v[buf], sem_g[buf])
            p = pltpu.async_copy(
                pe_hbm.at[pl.ds(t0 + s * STEP_POS, STEP_POS)],
                pe_v[buf], sem_p[buf])
            return g, p

        def compute(buf):
            def row_body(r, _):
                for c in range(vregs_per_row):
                    sl = pl.ds(c * LANES, LANES)
                    p = pe_v[buf][r, sl]
                    for b in range(batch):
                        row = b * STEP_POS + r
                        rows_v[buf][row, sl] = rows_v[buf][row, sl] * scale + p
                return 0
            lax.fori_loop(0, STEP_POS, row_body, 0)

        def start_stores(s, buf):
            st = []
            for b in range(batch):
                st.append(pltpu.async_copy(
                    rows_v[buf].at[pl.ds(b * STEP_POS, STEP_POS)],
                    out_hbm.at[pl.ds(b * seq_len + t0 + s * STEP_POS,
                                     STEP_POS)],
                    sem_s[buf]))
            return st

        inflight = [None, None]
        stores = [None, None]
        inflight[0] = start_step(0, 0)
        for s in range(steps):
            buf = s % 2
            if s + 1 < steps:
                nbuf = (s + 1) % 2
                if stores[nbuf] is not None:
                    for st in stores[nbuf]:
                        st.wait()
                    stores[nbuf] = None
                inflight[nbuf] = start_step(s + 1, nbuf)
            g, p = inflight[buf]
            g.wait()
            p.wait()
            compute(buf)
            stores[buf] = start_stores(s, buf)
        for side in stores:
            if side is not None:
                for st in side:
                    st.wait()

    return k(x_r, pe, table)


def kernel(x, table):
    batch, seq_len = x.shape
    chunk = seq_len // NUM_WORKERS
    steps = chunk // STEP_POS
    x_r = jnp.transpose(
        x.reshape(batch, NUM_WORKERS, steps, STEP_POS),
        (1, 2, 0, 3)).reshape(NUM_WORKERS * steps, batch * STEP_POS)
    pe = jnp.asarray(_pe_table(seq_len))
    out = _sc_embed(x_r, pe, table, batch=batch, seq_len=seq_len)
    return out.reshape(batch, seq_len, D_MODEL)
